# baseline (speedup 1.0000x reference)
import math

import jax
import jax.numpy as jnp
import numpy as np
from jax.experimental import pallas as pl
from jax.experimental.pallas import tpu as pltpu

_NUM_CLASS = 40
_BN_SCALE = float(1.0 / np.sqrt(1.0 + 1e-5))
_CFG = [(3, 32, 8, 1, -1), (32, 64, 8, 2, -1), (64, 96, 8, 4, -1),
        (96, 128, 12, 4, 120), (128, 160, 12, 6, 120)]
_N0 = 1024
_P4 = 120
_LB = 128
_PTS_OFF = 125

_SAMPLE_IDX = np.sort(np.random.RandomState(1234 + 3).choice(_N0, _P4, replace=False))
_S4_ONEHOT = np.zeros((_P4, _N0), np.float32)
_S4_ONEHOT[np.arange(_P4), _SAMPLE_IDX] = 1.0


def _layer_dims(C_in, C_out):
    C_half = C_out // 2
    C_mid = C_out // 4
    dm = min(int(math.ceil(C_out / C_in)), 4)
    Cc = C_mid + C_half
    return C_half, C_mid, dm, Cc


def _relu(x):
    return jnp.maximum(x, 0.0)


def _tl_init(dref, cmref, Pn):
    cs0 = [dref[0:Pn, c * _LB:(c + 1) * _LB] for c in range(8)]
    cmref[...] = jnp.concatenate(
        [jnp.min(c_, axis=1, keepdims=True) for c_ in cs0], axis=1)


def _tl_step(s, dref, cmref, paref, Pn, SL):
    iota8 = jax.lax.broadcasted_iota(jnp.int32, (Pn, 8), 1)
    iotaI = jax.lax.broadcasted_iota(jnp.int32, (Pn, _LB), 1)
    iotaS = jax.lax.broadcasted_iota(jnp.int32, (Pn, SL), 1)
    cm = cmref[...]
    m = jnp.min(cm, axis=1, keepdims=True)
    wc = jnp.min(jnp.where(cm <= m, iota8, 8), axis=1, keepdims=True)
    cs = [dref[0:Pn, c * _LB:(c + 1) * _LB] for c in range(8)]
    dwin = cs[0]
    for c in range(1, 8):
        dwin = jnp.where(wc == c, cs[c], dwin)
    ii = jnp.min(jnp.where(dwin <= m, iotaI, _LB), axis=1, keepdims=True)
    oh_in = iotaI == ii
    dwin_new = jnp.where(oh_in, 1e30, dwin)
    for c in range(8):
        dref[0:Pn, c * _LB:(c + 1) * _LB] = jnp.where(wc == c, dwin_new, cs[c])
    gidx = wc * _LB + ii
    paref[...] = jnp.where(iotaS == s, gidx.astype(jnp.float32), paref[...])
    newmin = jnp.min(dwin_new, axis=1, keepdims=True)
    cmref[...] = jnp.where(iota8 == wc, newmin, cm)


def _simple_step(s, dref, paref, Pn, NL, SL):
    iotaI = jax.lax.broadcasted_iota(jnp.int32, (Pn, NL), 1)
    iotaS = jax.lax.broadcasted_iota(jnp.int32, (Pn, SL), 1)
    d = dref[...]
    m = jnp.min(d, axis=1, keepdims=True)
    idx = jnp.min(jnp.where(d <= m, iotaI, NL), axis=1, keepdims=True)
    dref[...] = jnp.where(iotaI == idx, 1e30, d)
    paref[...] = jnp.where(iotaS == s, idx.astype(jnp.float32), paref[...])


def _body(pts_ref, ptsT_ref, ptsT5_ref, fts_ref, *refs):
    (out_ref, dref, gref, cmsh, pash, d4ref, cm4, pa4ref,
     d5ref, pa5ref) = refs[-10:]
    wrefs = refs[:-10]
    s4_ref = wrefs[75]
    fcn = wrefs[76:82]

    p_pts = pts_ref[0]
    p_fts = fts_ref[0]
    p_ptsT = ptsT_ref[0]
    pT5 = ptsT5_ref[0]

    rr0 = jnp.sum(p_ptsT * p_ptsT, axis=0, keepdims=True)
    rq0 = jnp.sum(p_pts * p_pts, axis=1, keepdims=True)
    dots0 = jax.lax.dot_general(
        p_pts, p_ptsT, (((1,), (0,)), ((), ())),
        preferred_element_type=jnp.float32)
    dref[...] = (rq0 - 2.0 * dots0) + rr0
    iota0 = jax.lax.broadcasted_iota(jnp.int32, (_N0, _N0), 1)

    rep4 = s4_ref[...] @ p_pts
    rq4 = jnp.sum(rep4 * rep4, axis=1, keepdims=True)
    dots4 = jax.lax.dot_general(
        rep4, p_ptsT, (((1,), (0,)), ((), ())),
        preferred_element_type=jnp.float32)
    d4ref[...] = (rq4 - 2.0 * dots4) + rr0

    rr5 = jnp.sum(pT5 * pT5, axis=0, keepdims=True)
    dots5 = jax.lax.dot_general(
        rep4, pT5, (((1,), (0,)), ((), ())),
        preferred_element_type=jnp.float32)
    d5ref[...] = jnp.full((_P4, _LB), 1e30, jnp.float32)
    d5ref[0:_P4, 0:_P4] = (rq4 - 2.0 * dots5) + rr5

    _tl_init(dref, cmsh, _N0)
    _tl_init(d4ref, cm4, _P4)

    for s in range(30):
        _tl_step(s, dref, cmsh, pash, _N0, 32)
        for j in range(2):
            if 2 * s + j < 46:
                _tl_step(2 * s + j, d4ref, cm4, pa4ref, _P4, 64)
        for j in range(3):
            if 3 * s + j < 68:
                _simple_step(3 * s + j, d5ref, pa5ref, _P4, _LB, _LB)

    for i, (C_in, C_out, K, D, P) in enumerate(_CFG):
        (lift_W, lift_b, d1bd, d1bt, d2bdp, d2btp, xtW, xt_b,
         xtd1, xtd1b, xtd2, xtd2b, W2p, b2, Ep) = wrefs[15 * i:15 * (i + 1)]
        C_half, C_mid, dm, Cc = _layer_dims(C_in, C_out)
        N = p_pts.shape[0]

        fts_lift = _relu(p_fts @ lift_W[...] + lift_b[...])
        rep = rep4 if i == 3 else p_pts
        Pn = rep.shape[0]

        srcp = jnp.concatenate(
            [fts_lift, jnp.zeros((N, _PTS_OFF - C_half), jnp.float32), p_pts],
            axis=1)

        if i < 3:
            pa, iota_n = pash[...], iota0
        elif i == 3:
            pa = pa4ref[...]
            iota_n = jax.lax.broadcasted_iota(jnp.int32, (Pn, N), 1)
        else:
            pa = pa5ref[...]
            iota_n = jax.lax.broadcasted_iota(jnp.int32, (Pn, N), 1)

        for k in range(K):
            idxk = pa[:, 1 + k * D:2 + k * D].astype(jnp.int32)
            oh = jnp.where(iota_n == idxk, 1.0, 0.0)
            g = jax.lax.dot_general(
                oh, srcp, (((1,), (0,)), ((), ())),
                preferred_element_type=jnp.float32)
            gref[k, 0:Pn, :] = g

        pls = [gref[k, 0:Pn, _PTS_OFF:_PTS_OFF + 3] - rep
               for k in range(K)]
        pl_cat = jnp.concatenate(pls, axis=1)
        gcat = jnp.concatenate(
            [gref[k, 0:Pn, :] for k in range(K)], axis=1)

        f_cat = _relu(pl_cat @ d1bd[...] + d1bt[...])
        f_catp = _relu(f_cat @ d2bdp[...] + d2btp[...])
        catp = gcat + f_catp

        t = _relu(pl_cat @ xtW[...] + xt_b[...])
        t = _relu(t @ xtd1[...] + xtd1b[...])
        X = t @ xtd2[...] + xtd2b[...]

        acc = None
        for k in range(K):
            Xb = jax.lax.dot_general(
                X[:, k * K:(k + 1) * K], Ep[...],
                (((1,), (0,)), ((), ())),
                preferred_element_type=jnp.float32)
            prod = Xb * catp
            fxk = prod[:, 0:_LB]
            for l in range(1, K):
                fxk = fxk + prod[:, l * _LB:(l + 1) * _LB]
            part = jax.lax.dot_general(
                fxk, W2p[k * _LB:(k + 1) * _LB, :],
                (((1,), (0,)), ((), ())),
                preferred_element_type=jnp.float32)
            acc = part if acc is None else acc + part

        out = acc + b2[...]
        p_fts = _relu(out) * _BN_SCALE
        p_pts = rep
        if i == 3:
            p_ptsT = ptsT5_ref[0]

    f1_W, f1_b, f2_W, f2_b, f3_W, f3_b = fcn
    x = _relu(p_fts @ f1_W[...] + f1_b[...])
    x = _relu(x @ f2_W[...] + f2_b[...])
    logits = x @ f3_W[...] + f3_b[...]
    out_ref[...] = jnp.mean(logits, axis=0, keepdims=True)[None]


def _prep_layer(lp, C_in, C_out, K):
    C_half, C_mid, dm, Cc = _layer_dims(C_in, C_out)
    eyeK = jnp.eye(K, dtype=jnp.float32)
    d1bd = jnp.kron(eyeK, lp['d1_W'])
    d1bt = jnp.tile(lp['d1_b'], K)
    d2bdp = jnp.zeros((K * C_mid, K * _LB), jnp.float32)
    for l in range(K):
        d2bdp = d2bdp.at[l * C_mid:(l + 1) * C_mid,
                         l * _LB + C_half:l * _LB + C_half + C_mid].set(lp['d2_W'])
    d2btp = jnp.zeros((K * _LB,), jnp.float32)
    for l in range(K):
        d2btp = d2btp.at[l * _LB + C_half:l * _LB + C_half + C_mid].set(lp['d2_b'])
    xtW = lp['xt_conv_W'].transpose(1, 2, 0).reshape(3 * K, K * K)
    W2 = jnp.einsum('cmk,ocm->kco', lp['ec_dw_W'], lp['ec_pw_W'])
    W2p = jnp.concatenate(
        [W2[:, C_mid:, :], W2[:, :C_mid, :],
         jnp.zeros((K, _LB - Cc, C_out), jnp.float32)], axis=1)
    W2p = W2p.reshape(K * _LB, C_out)
    b2 = jnp.einsum('cm,ocm->o', lp['ec_dw_b'], lp['ec_pw_W'])
    pat = jnp.concatenate(
        [jnp.ones((_PTS_OFF,), jnp.float32),
         jnp.zeros((_LB - _PTS_OFF,), jnp.float32)])[None, :]
    Ep = jnp.kron(eyeK, pat)
    return [lp['lift_W'], lp['lift_b'], d1bd, d1bt, d2bdp, d2btp,
            xtW, lp['xt_conv_b'], lp['xt_d1_W'], lp['xt_d1_b'],
            lp['xt_d2_W'], lp['xt_d2_b'], W2p, b2, Ep]


def _full_spec(a):
    shp = a.shape
    return pl.BlockSpec(shp, lambda b, _r=len(shp): (0,) * _r)


def kernel(pts, fts, params):
    B = pts.shape[0]
    ptsT = pts.transpose(0, 2, 1)
    ptsT5 = ptsT[:, :, jnp.asarray(_SAMPLE_IDX)]

    weights = []
    for i, (C_in, C_out, K, D, P) in enumerate(_CFG):
        weights += _prep_layer(params['layers'][i], C_in, C_out, K)
    weights.append(jnp.asarray(_S4_ONEHOT))
    f = params['fcn']
    weights += [f['f1_W'], f['f1_b'], f['f2_W'], f['f2_b'], f['f3_W'], f['f3_b']]

    in_specs = [
        pl.BlockSpec((1, _N0, 3), lambda b: (b, 0, 0)),
        pl.BlockSpec((1, 3, _N0), lambda b: (b, 0, 0)),
        pl.BlockSpec((1, 3, _P4), lambda b: (b, 0, 0)),
        pl.BlockSpec((1, _N0, 3), lambda b: (b, 0, 0)),
    ] + [_full_spec(w) for w in weights]

    out = pl.pallas_call(
        _body,
        grid=(B,),
        in_specs=in_specs,
        out_specs=pl.BlockSpec((1, 1, _NUM_CLASS), lambda b: (b, 0, 0)),
        out_shape=jax.ShapeDtypeStruct((B, 1, _NUM_CLASS), jnp.float32),
        scratch_shapes=[
            pltpu.VMEM((_N0, _N0), jnp.float32),
            pltpu.VMEM((12, _N0, _LB), jnp.float32),
            pltpu.VMEM((_N0, 8), jnp.float32),
            pltpu.VMEM((_N0, 32), jnp.float32),
            pltpu.VMEM((_P4, _N0), jnp.float32),
            pltpu.VMEM((_P4, 8), jnp.float32),
            pltpu.VMEM((_P4, 64), jnp.float32),
            pltpu.VMEM((_P4, _LB), jnp.float32),
            pltpu.VMEM((_P4, _LB), jnp.float32),
        ],
        compiler_params=pltpu.CompilerParams(
            vmem_limit_bytes=100 * 1024 * 1024),
    )(pts, ptsT, ptsT5, fts, *weights)
    return out[:, 0, :]


# speedup vs baseline: 1.0475x; 1.0475x over previous
import math

import jax
import jax.numpy as jnp
import numpy as np
from jax.experimental import pallas as pl
from jax.experimental.pallas import tpu as pltpu

_NUM_CLASS = 40
_BN_SCALE = float(1.0 / np.sqrt(1.0 + 1e-5))
_CFG = [(3, 32, 8, 1, -1), (32, 64, 8, 2, -1), (64, 96, 8, 4, -1),
        (96, 128, 12, 4, 120), (128, 160, 12, 6, 120)]
_N0 = 1024
_P4 = 120
_LB = 128
_PTS_OFF = 125

_SAMPLE_IDX = np.sort(np.random.RandomState(1234 + 3).choice(_N0, _P4, replace=False))
_S4_ONEHOT = np.zeros((_P4, _N0), np.float32)
_S4_ONEHOT[np.arange(_P4), _SAMPLE_IDX] = 1.0


def _layer_dims(C_in, C_out):
    C_half = C_out // 2
    C_mid = C_out // 4
    dm = min(int(math.ceil(C_out / C_in)), 4)
    Cc = C_mid + C_half
    return C_half, C_mid, dm, Cc


def _relu(x):
    return jnp.maximum(x, 0.0)


def _tl_init(dref, cmref, Pn):
    cs0 = [dref[0:Pn, c * _LB:(c + 1) * _LB] for c in range(8)]
    cmref[...] = jnp.concatenate(
        [jnp.min(c_, axis=1, keepdims=True) for c_ in cs0], axis=1)


def _tl_step(s, dref, cmref, paref, Pn, SL):
    iota8 = jax.lax.broadcasted_iota(jnp.int32, (Pn, 8), 1)
    iotaI = jax.lax.broadcasted_iota(jnp.int32, (Pn, _LB), 1)
    iotaS = jax.lax.broadcasted_iota(jnp.int32, (Pn, SL), 1)
    cm = cmref[...]
    m = jnp.min(cm, axis=1, keepdims=True)
    wc = jnp.min(jnp.where(cm <= m, iota8, 8), axis=1, keepdims=True)
    cs = [dref[0:Pn, c * _LB:(c + 1) * _LB] for c in range(8)]
    dwin = cs[0]
    for c in range(1, 8):
        dwin = jnp.where(wc == c, cs[c], dwin)
    ii = jnp.min(jnp.where(dwin <= m, iotaI, _LB), axis=1, keepdims=True)
    oh_in = iotaI == ii
    dwin_new = jnp.where(oh_in, 1e30, dwin)
    for c in range(8):
        dref[0:Pn, c * _LB:(c + 1) * _LB] = jnp.where(wc == c, dwin_new, cs[c])
    gidx = wc * _LB + ii
    paref[...] = jnp.where(iotaS == s, gidx.astype(jnp.float32), paref[...])
    newmin = jnp.min(dwin_new, axis=1, keepdims=True)
    cmref[...] = jnp.where(iota8 == wc, newmin, cm)


def _simple_step(s, dref, paref, Pn, NL, SL):
    iotaI = jax.lax.broadcasted_iota(jnp.int32, (Pn, NL), 1)
    iotaS = jax.lax.broadcasted_iota(jnp.int32, (Pn, SL), 1)
    d = dref[...]
    m = jnp.min(d, axis=1, keepdims=True)
    idx = jnp.min(jnp.where(d <= m, iotaI, NL), axis=1, keepdims=True)
    dref[...] = jnp.where(iotaI == idx, 1e30, d)
    paref[...] = jnp.where(iotaS == s, idx.astype(jnp.float32), paref[...])


def _body(pts_ref, ptsT_ref, ptsT5_ref, fts_ref, *refs):
    (out_ref, dref, gref, cmsh, pash, d4ref, cm4, pa4ref,
     d5ref, pa5ref) = refs[-10:]
    wrefs = refs[:-10]
    s4_ref = wrefs[75]
    fcn = wrefs[76:82]

    p_pts = pts_ref[0]
    p_fts = fts_ref[0]
    p_ptsT = ptsT_ref[0]
    pT5 = ptsT5_ref[0]

    rr0 = jnp.sum(p_ptsT * p_ptsT, axis=0, keepdims=True)
    rq0 = jnp.sum(p_pts * p_pts, axis=1, keepdims=True)
    dots0 = jax.lax.dot_general(
        p_pts, p_ptsT, (((1,), (0,)), ((), ())),
        preferred_element_type=jnp.float32)
    dref[...] = (rq0 - 2.0 * dots0) + rr0
    iota0 = jax.lax.broadcasted_iota(jnp.int32, (_N0, _N0), 1)

    rep4 = s4_ref[...] @ p_pts
    rq4 = jnp.sum(rep4 * rep4, axis=1, keepdims=True)
    dots4 = jax.lax.dot_general(
        rep4, p_ptsT, (((1,), (0,)), ((), ())),
        preferred_element_type=jnp.float32)
    d4ref[...] = (rq4 - 2.0 * dots4) + rr0

    rr5 = jnp.sum(pT5 * pT5, axis=0, keepdims=True)
    dots5 = jax.lax.dot_general(
        rep4, pT5, (((1,), (0,)), ((), ())),
        preferred_element_type=jnp.float32)
    d5ref[...] = jnp.full((_P4, _LB), 1e30, jnp.float32)
    d5ref[0:_P4, 0:_P4] = (rq4 - 2.0 * dots5) + rr5

    _tl_init(dref, cmsh, _N0)
    _tl_init(d4ref, cm4, _P4)

    def uni_step(s, carry):
        _tl_step(s, dref, cmsh, pash, _N0, 32)
        for j in range(2):
            _tl_step(2 * s + j, d4ref, cm4, pa4ref, _P4, 64)
        for j in range(3):
            _simple_step(3 * s + j, d5ref, pa5ref, _P4, _LB, _LB)
        return carry

    jax.lax.fori_loop(0, 30, uni_step, 0)

    for i, (C_in, C_out, K, D, P) in enumerate(_CFG):
        (lift_W, lift_b, d1bd, d1bt, d2bdp, d2btp, xtW, xt_b,
         xtd1, xtd1b, xtd2, xtd2b, W2p, b2, Ep) = wrefs[15 * i:15 * (i + 1)]
        C_half, C_mid, dm, Cc = _layer_dims(C_in, C_out)
        N = p_pts.shape[0]

        fts_lift = _relu(p_fts @ lift_W[...] + lift_b[...])
        rep = rep4 if i == 3 else p_pts
        Pn = rep.shape[0]

        srcp = jnp.concatenate(
            [fts_lift, jnp.zeros((N, _PTS_OFF - C_half), jnp.float32), p_pts],
            axis=1)

        if i < 3:
            pa, iota_n = pash[...], iota0
        elif i == 3:
            pa = pa4ref[...]
            iota_n = jax.lax.broadcasted_iota(jnp.int32, (Pn, N), 1)
        else:
            pa = pa5ref[...]
            iota_n = jax.lax.broadcasted_iota(jnp.int32, (Pn, N), 1)

        for k in range(K):
            idxk = pa[:, 1 + k * D:2 + k * D].astype(jnp.int32)
            oh = jnp.where(iota_n == idxk, 1.0, 0.0)
            g = jax.lax.dot_general(
                oh, srcp, (((1,), (0,)), ((), ())),
                preferred_element_type=jnp.float32)
            gref[k, 0:Pn, :] = g

        pls = [gref[k, 0:Pn, _PTS_OFF:_PTS_OFF + 3] - rep
               for k in range(K)]
        pl_cat = jnp.concatenate(pls, axis=1)
        gcat = jnp.concatenate(
            [gref[k, 0:Pn, :] for k in range(K)], axis=1)

        f_cat = _relu(pl_cat @ d1bd[...] + d1bt[...])
        f_catp = _relu(f_cat @ d2bdp[...] + d2btp[...])
        catp = gcat + f_catp

        t = _relu(pl_cat @ xtW[...] + xt_b[...])
        t = _relu(t @ xtd1[...] + xtd1b[...])
        X = t @ xtd2[...] + xtd2b[...]

        acc = None
        for k in range(K):
            Xb = jax.lax.dot_general(
                X[:, k * K:(k + 1) * K], Ep[...],
                (((1,), (0,)), ((), ())),
                preferred_element_type=jnp.float32)
            prod = Xb * catp
            fxk = prod[:, 0:_LB]
            for l in range(1, K):
                fxk = fxk + prod[:, l * _LB:(l + 1) * _LB]
            part = jax.lax.dot_general(
                fxk, W2p[k * _LB:(k + 1) * _LB, :],
                (((1,), (0,)), ((), ())),
                preferred_element_type=jnp.float32)
            acc = part if acc is None else acc + part

        out = acc + b2[...]
        p_fts = _relu(out) * _BN_SCALE
        p_pts = rep
        if i == 3:
            p_ptsT = ptsT5_ref[0]

    f1_W, f1_b, f2_W, f2_b, f3_W, f3_b = fcn
    x = _relu(p_fts @ f1_W[...] + f1_b[...])
    x = _relu(x @ f2_W[...] + f2_b[...])
    logits = x @ f3_W[...] + f3_b[...]
    out_ref[...] = jnp.mean(logits, axis=0, keepdims=True)[None]


def _prep_layer(lp, C_in, C_out, K):
    C_half, C_mid, dm, Cc = _layer_dims(C_in, C_out)
    eyeK = jnp.eye(K, dtype=jnp.float32)
    d1bd = jnp.kron(eyeK, lp['d1_W'])
    d1bt = jnp.tile(lp['d1_b'], K)
    d2bdp = jnp.zeros((K * C_mid, K * _LB), jnp.float32)
    for l in range(K):
        d2bdp = d2bdp.at[l * C_mid:(l + 1) * C_mid,
                         l * _LB + C_half:l * _LB + C_half + C_mid].set(lp['d2_W'])
    d2btp = jnp.zeros((K * _LB,), jnp.float32)
    for l in range(K):
        d2btp = d2btp.at[l * _LB + C_half:l * _LB + C_half + C_mid].set(lp['d2_b'])
    xtW = lp['xt_conv_W'].transpose(1, 2, 0).reshape(3 * K, K * K)
    W2 = jnp.einsum('cmk,ocm->kco', lp['ec_dw_W'], lp['ec_pw_W'])
    W2p = jnp.concatenate(
        [W2[:, C_mid:, :], W2[:, :C_mid, :],
         jnp.zeros((K, _LB - Cc, C_out), jnp.float32)], axis=1)
    W2p = W2p.reshape(K * _LB, C_out)
    b2 = jnp.einsum('cm,ocm->o', lp['ec_dw_b'], lp['ec_pw_W'])
    pat = jnp.concatenate(
        [jnp.ones((_PTS_OFF,), jnp.float32),
         jnp.zeros((_LB - _PTS_OFF,), jnp.float32)])[None, :]
    Ep = jnp.kron(eyeK, pat)
    return [lp['lift_W'], lp['lift_b'], d1bd, d1bt, d2bdp, d2btp,
            xtW, lp['xt_conv_b'], lp['xt_d1_W'], lp['xt_d1_b'],
            lp['xt_d2_W'], lp['xt_d2_b'], W2p, b2, Ep]


def _full_spec(a):
    shp = a.shape
    return pl.BlockSpec(shp, lambda b, _r=len(shp): (0,) * _r)


def kernel(pts, fts, params):
    B = pts.shape[0]
    ptsT = pts.transpose(0, 2, 1)
    ptsT5 = ptsT[:, :, jnp.asarray(_SAMPLE_IDX)]

    weights = []
    for i, (C_in, C_out, K, D, P) in enumerate(_CFG):
        weights += _prep_layer(params['layers'][i], C_in, C_out, K)
    weights.append(jnp.asarray(_S4_ONEHOT))
    f = params['fcn']
    weights += [f['f1_W'], f['f1_b'], f['f2_W'], f['f2_b'], f['f3_W'], f['f3_b']]

    in_specs = [
        pl.BlockSpec((1, _N0, 3), lambda b: (b, 0, 0)),
        pl.BlockSpec((1, 3, _N0), lambda b: (b, 0, 0)),
        pl.BlockSpec((1, 3, _P4), lambda b: (b, 0, 0)),
        pl.BlockSpec((1, _N0, 3), lambda b: (b, 0, 0)),
    ] + [_full_spec(w) for w in weights]

    out = pl.pallas_call(
        _body,
        grid=(B,),
        in_specs=in_specs,
        out_specs=pl.BlockSpec((1, 1, _NUM_CLASS), lambda b: (b, 0, 0)),
        out_shape=jax.ShapeDtypeStruct((B, 1, _NUM_CLASS), jnp.float32),
        scratch_shapes=[
            pltpu.VMEM((_N0, _N0), jnp.float32),
            pltpu.VMEM((12, _N0, _LB), jnp.float32),
            pltpu.VMEM((_N0, 8), jnp.float32),
            pltpu.VMEM((_N0, 32), jnp.float32),
            pltpu.VMEM((_P4, _N0), jnp.float32),
            pltpu.VMEM((_P4, 8), jnp.float32),
            pltpu.VMEM((_P4, 64), jnp.float32),
            pltpu.VMEM((_P4, _LB), jnp.float32),
            pltpu.VMEM((_P4, _LB), jnp.float32),
        ],
        compiler_params=pltpu.CompilerParams(
            vmem_limit_bytes=100 * 1024 * 1024),
    )(pts, ptsT, ptsT5, fts, *weights)
    return out[:, 0, :]
